# baseline (speedup 1.0000x reference)
import functools

import jax
import jax.numpy as jnp
from jax import lax
from jax.experimental import pallas as pl
from jax.experimental.pallas import tpu as pltpu
from jax.experimental.pallas import tpu_sc as plsc

N_CLASS_K = 26
CPAD = 32
LANES = 16
NC, NS = 2, 16
NW = NC * NS
ACC = CPAD * LANES
CHUNKS = (32, 64, 128, 144, 144)
NQ = len(CHUNKS)


def _tree_sum(vs):
    vs = list(vs)
    while len(vs) > 1:
        vs = [a + b for a, b in zip(vs[0::2], vs[1::2])]
    return vs[0]


def _sc_partials(x, label):
    n, d = x.shape
    rw = n // NW
    kc = d // LANES

    mesh = plsc.VectorSubcoreMesh(core_axis_name="c", subcore_axis_name="s")

    @functools.partial(
        pl.kernel,
        out_type=jax.ShapeDtypeStruct((NW * 3 * ACC,), jnp.float32),
        mesh=mesh,
        compiler_params=pltpu.CompilerParams(needs_layout_passes=False),
        scratch_types=[
            pltpu.VMEM((rw, d), jnp.float32),
            pltpu.VMEM((rw,), jnp.int32),
            pltpu.VMEM((ACC,), jnp.float32),
            pltpu.VMEM((ACC,), jnp.float32),
            pltpu.VMEM((ACC,), jnp.float32),
            pltpu.SemaphoreType.DMA,
            pltpu.SemaphoreType.DMA,
            pltpu.SemaphoreType.DMA,
            pltpu.SemaphoreType.DMA,
            pltpu.SemaphoreType.DMA,
            pltpu.SemaphoreType.DMA,
        ],
    )
    def sc_k(x_hbm, lab_hbm, out_hbm, x_v, lab_v, sum_v, sq_v, cnt_v,
             sem0, sem1, sem2, sem3, sem4, lsem):
        wid = lax.axis_index("s") * NC + lax.axis_index("c")
        base = wid * rw
        starts = [sum(CHUNKS[:q]) for q in range(NQ)]
        sems = (sem0, sem1, sem2, sem3, sem4)

        def chunk_copy(q):
            return pltpu.make_async_copy(
                x_hbm.at[pl.ds(base + starts[q], CHUNKS[q])],
                x_v.at[pl.ds(starts[q], CHUNKS[q])], sems[q])

        lcp = pltpu.make_async_copy(
            lab_hbm.at[pl.ds(base, rw)], lab_v, lsem)
        lcp.start()
        chunk_copy(0).start()
        chunk_copy(1).start()
        zero = jnp.zeros((LANES,), jnp.float32)
        for i in range(CPAD):
            sum_v[pl.ds(i * LANES, LANES)] = zero
            sq_v[pl.ds(i * LANES, LANES)] = zero
            cnt_v[pl.ds(i * LANES, LANES)] = zero
        lcp.wait()
        chunk_copy(0).wait()
        chunk_copy(2).start()
        lane_iota = lax.iota(jnp.int32, LANES)
        ones = jnp.ones((LANES,), jnp.float32)

        def group_body(g, carry):
            for q in range(1, NQ):
                @pl.when(g == starts[q] // LANES)
                def _(q=q):
                    chunk_copy(q).wait()
                    if q + 2 < NQ:
                        chunk_copy(q + 2).start()

            labs = lab_v[pl.ds(g * LANES, LANES)]
            results = []
            for j in range(LANES):
                r = g * LANES + j
                vs = [x_v[r, pl.ds(k * LANES, LANES)] for k in range(kc)]
                results.append((j, _tree_sum(vs),
                                _tree_sum([v * v for v in vs])))
            plsc.addupdate_scatter(cnt_v, [labs * LANES + lane_iota], ones)
            for j, acc, sq in results:
                idx = labs[j] * LANES + lane_iota
                plsc.addupdate_scatter(sum_v, [idx], acc)
                plsc.addupdate_scatter(sq_v, [idx], sq)
            return carry

        lax.fori_loop(0, rw // LANES, group_body, 0)
        obase = wid * 3 * ACC
        pltpu.sync_copy(sum_v, out_hbm.at[pl.ds(obase, ACC)])
        pltpu.sync_copy(sq_v, out_hbm.at[pl.ds(obase + ACC, ACC)])
        pltpu.sync_copy(cnt_v, out_hbm.at[pl.ds(obase + 2 * ACC, ACC)])

    return sc_k(x, label)


def _tc_loss(partials, d):
    rows_per_stat = ACC // 128
    g_per_row = 128 // LANES

    def tc_body(p_ref, o_ref):
        p = p_ref[...]
        s = jnp.sum(p.reshape(NW, 3 * rows_per_stat, 128), axis=0)
        j = lax.broadcasted_iota(jnp.int32, (128, g_per_row), 0)
        c = lax.broadcasted_iota(jnp.int32, (128, g_per_row), 1)
        sel = (j // LANES == c).astype(jnp.float32)
        st = jnp.dot(s, sel, preferred_element_type=jnp.float32)
        sum_x = st[0:4, :]
        sum_sq = st[4:8, :]
        cnt = st[8:12, :] * d
        var = (sum_sq - sum_x * sum_x / cnt) / (cnt - 1.0)
        ca = lax.broadcasted_iota(jnp.int32, (rows_per_stat, g_per_row), 0)
        cb = lax.broadcasted_iota(jnp.int32, (rows_per_stat, g_per_row), 1)
        valid = (ca * g_per_row + cb) < N_CLASS_K
        loss = jnp.sum(jnp.where(valid, jnp.sqrt(var), 0.0), keepdims=True)
        o_ref[...] = loss

    out = pl.pallas_call(
        tc_body,
        out_shape=jax.ShapeDtypeStruct((1, 1), jnp.float32),
    )(partials.reshape(NW * 3 * ACC // 128, 128))
    return out[0, 0]


def kernel(x, label, n_class):
    del n_class
    partials = _sc_partials(x, label.astype(jnp.int32))
    return _tc_loss(partials, x.shape[1])


# speedup vs baseline: 1.0246x; 1.0246x over previous
import functools

import jax
import jax.numpy as jnp
from jax import lax
from jax.experimental import pallas as pl
from jax.experimental.pallas import tpu as pltpu
from jax.experimental.pallas import tpu_sc as plsc

N_CLASS_K = 26
CPAD = 32
LANES = 16
NC, NS = 2, 16
NW = NC * NS
ACC = CPAD * LANES
NQ = 4
NBUF = 4


def _tree_sum(vs):
    vs = list(vs)
    while len(vs) > 1:
        vs = [a + b for a, b in zip(vs[0::2], vs[1::2])]
    return vs[0]


def _sc_partials(x, label):
    n, d = x.shape
    rw = n // NW
    kc = d // LANES

    mesh = plsc.VectorSubcoreMesh(core_axis_name="c", subcore_axis_name="s")

    @functools.partial(
        pl.kernel,
        out_type=jax.ShapeDtypeStruct((NW * 3 * ACC,), jnp.float32),
        mesh=mesh,
        compiler_params=pltpu.CompilerParams(needs_layout_passes=False),
        scratch_types=[
            pltpu.VMEM((rw, d), jnp.float32),
            pltpu.VMEM((rw,), jnp.int32),
            pltpu.VMEM((ACC,), jnp.float32),
            pltpu.VMEM((ACC,), jnp.float32),
            pltpu.VMEM((ACC,), jnp.float32),
            pltpu.SemaphoreType.DMA,
            pltpu.SemaphoreType.DMA,
            pltpu.SemaphoreType.DMA,
            pltpu.SemaphoreType.DMA,
            pltpu.SemaphoreType.DMA,
        ],
    )
    def sc_k(x_hbm, lab_hbm, out_hbm, x_v, lab_v, sum_v, sq_v, cnt_v,
             sem0, sem1, sem2, sem3, lsem):
        wid = lax.axis_index("s") * NC + lax.axis_index("c")
        base = wid * rw
        qrows = rw // NQ
        gpq = qrows // LANES
        sems = (sem0, sem1, sem2, sem3)

        def chunk_copy(q):
            return pltpu.make_async_copy(
                x_hbm.at[pl.ds(base + q * qrows, qrows)],
                x_v.at[pl.ds(q * qrows, qrows)], sems[q])

        lcp = pltpu.make_async_copy(
            lab_hbm.at[pl.ds(base, rw)], lab_v, lsem)
        lcp.start()
        chunk_copy(0).start()
        chunk_copy(1).start()
        zero = jnp.zeros((LANES,), jnp.float32)
        for i in range(CPAD):
            sum_v[pl.ds(i * LANES, LANES)] = zero
            sq_v[pl.ds(i * LANES, LANES)] = zero
            cnt_v[pl.ds(i * LANES, LANES)] = zero
        lcp.wait()
        chunk_copy(0).wait()
        chunk_copy(2).start()
        lane_iota = lax.iota(jnp.int32, LANES)
        ones = jnp.ones((LANES,), jnp.float32)

        def group_body(g, carry):
            for q in range(1, NQ):
                @pl.when(g == q * gpq)
                def _(q=q):
                    chunk_copy(q).wait()
                    if q + 2 < NQ:
                        chunk_copy(q + 2).start()

            labs = lab_v[pl.ds(g * LANES, LANES)]
            results = []
            for j in range(LANES):
                r = g * LANES + j
                vs = [x_v[r, pl.ds(k * LANES, LANES)] for k in range(kc)]
                results.append((j, _tree_sum(vs),
                                _tree_sum([v * v for v in vs])))
            plsc.addupdate_scatter(cnt_v, [labs * LANES + lane_iota], ones)
            for j, acc, sq in results:
                idx = labs[j] * LANES + lane_iota
                plsc.addupdate_scatter(sum_v, [idx], acc)
                plsc.addupdate_scatter(sq_v, [idx], sq)
            return carry

        lax.fori_loop(0, rw // LANES, group_body, 0)
        obase = wid * 3 * ACC
        pltpu.sync_copy(sum_v, out_hbm.at[pl.ds(obase, ACC)])
        pltpu.sync_copy(sq_v, out_hbm.at[pl.ds(obase + ACC, ACC)])
        pltpu.sync_copy(cnt_v, out_hbm.at[pl.ds(obase + 2 * ACC, ACC)])

    return sc_k(x, label)


def _tc_loss(partials, d):
    rows_per_stat = ACC // 128
    g_per_row = 128 // LANES

    def tc_body(p_ref, o_ref):
        p = p_ref[...]
        s = jnp.sum(p.reshape(NW, 3 * rows_per_stat, 128), axis=0)
        j = lax.broadcasted_iota(jnp.int32, (128, g_per_row), 0)
        c = lax.broadcasted_iota(jnp.int32, (128, g_per_row), 1)
        sel = (j // LANES == c).astype(jnp.float32)
        st = jnp.dot(s, sel, preferred_element_type=jnp.float32)
        sum_x = st[0:4, :]
        sum_sq = st[4:8, :]
        cnt = st[8:12, :] * d
        var = (sum_sq - sum_x * sum_x / cnt) / (cnt - 1.0)
        ca = lax.broadcasted_iota(jnp.int32, (rows_per_stat, g_per_row), 0)
        cb = lax.broadcasted_iota(jnp.int32, (rows_per_stat, g_per_row), 1)
        valid = (ca * g_per_row + cb) < N_CLASS_K
        loss = jnp.sum(jnp.where(valid, jnp.sqrt(var), 0.0), keepdims=True)
        o_ref[...] = loss

    out = pl.pallas_call(
        tc_body,
        out_shape=jax.ShapeDtypeStruct((1, 1), jnp.float32),
    )(partials.reshape(NW * 3 * ACC // 128, 128))
    return out[0, 0]


def kernel(x, label, n_class):
    del n_class
    partials = _sc_partials(x, label.astype(jnp.int32))
    return _tc_loss(partials, x.shape[1])
